# baseline (speedup 1.0000x reference)
import dataclasses
import functools

import jax
import jax.numpy as jnp
from jax import lax
from jax.experimental import pallas as pl
from jax.experimental.pallas import tpu as pltpu
from jax.experimental.pallas import tpu_sc as plsc

NODE_N = 10000
POD_N = 30000
SVC_N = 10000
T = 8
D = 64
H = 64
TH = T * H

NC = 2
NS = 16
L = 16

BATCH = 128
PADR = 64
HPAD = 80
CROWS = 160
CAP = NC * NS * CROWS
EBLK = 2048
GB = 16
FB = 48

_mesh = plsc.VectorSubcoreMesh(core_axis_name="core", subcore_axis_name="subcore")

_sc_params = pltpu.CompilerParams()
if "needs_layout_passes" in pltpu.CompilerParams.__dataclass_fields__:
    _sc_params = dataclasses.replace(_sc_params, needs_layout_passes=False)


def _cpad(n):
    return ((n + PADR + 255) // 256) * 256


def _accr(n):
    return ((n + PADR + CAP - 1) // CAP) * CAP


def _pad_edges(e, n_src, n_dst):
    E = e.shape[1]
    ep = ((E + NS * BATCH - 1) // (NS * BATCH)) * (NS * BATCH)
    fan = (jnp.arange(ep - E, dtype=jnp.int32) % PADR)
    src = jnp.concatenate([e[0].astype(jnp.int32), n_src + fan])
    dst = jnp.concatenate([e[1].astype(jnp.int32), n_dst + fan])
    return src, dst


def _degree_kernel(edges):
    rels = ["in", "ni", "ii", "sc"]
    core_of = {"in": 0, "ni": 0, "ii": 1, "sc": 1}
    CMAX = _cpad(POD_N)
    DBLK = 128

    out_type = []
    for r in rels:
        _, _, n_s, n_d = edges[r]
        out_type.append(jax.ShapeDtypeStruct((n_s,), jnp.float32))
        out_type.append(jax.ShapeDtypeStruct((n_d,), jnp.float32))

    @functools.partial(
        pl.kernel, out_type=out_type, mesh=_mesh, compiler_params=_sc_params,
        scratch_types=[
            pltpu.VMEM((DBLK,), jnp.int32),
            pltpu.VMEM((DBLK,), jnp.int32),
            pltpu.VMEM((CMAX,), jnp.float32),
            pltpu.VMEM((CMAX,), jnp.float32),
            pltpu.VMEM((CMAX // 16,), jnp.float32),
            pltpu.VMEM((CMAX // 16,), jnp.float32),
            pltpu.VMEM_SHARED((NS * CMAX,), jnp.float32),
        ],
    )
    def body(*refs):
        in_refs = refs[: 2 * len(rels)]
        o_refs = refs[2 * len(rels): 4 * len(rels)]
        sblk, dblk, cnt_a, cnt_b, ssum, srd, shared = refs[4 * len(rels):]
        cid = lax.axis_index("core")
        sid = lax.axis_index("subcore")
        ones = jnp.full((L,), 1.0, jnp.float32)
        zeros = jnp.zeros((L,), jnp.float32)

        for ri, r in enumerate(rels):
            _, _, n_s, n_d = edges[r]
            src_r = in_refs[2 * ri]
            dst_r = in_refs[2 * ri + 1]
            edges_per_tile = src_r.shape[0] // NS
            nfull = edges_per_tile // DBLK
            tail = edges_per_tile % DBLK

            @pl.when(cid == core_of[r])
            def _():
                @pl.loop(0, _cpad(n_s) // 16)
                def _(i):
                    cnt_a[pl.ds(i * 16, 16)] = zeros

                @pl.loop(0, _cpad(n_d) // 16)
                def _(i):
                    cnt_b[pl.ds(i * 16, 16)] = zeros

                def hist_block(base, ne):
                    pltpu.sync_copy(src_r.at[pl.ds(base, ne)], sblk.at[pl.ds(0, ne)])
                    pltpu.sync_copy(dst_r.at[pl.ds(base, ne)], dblk.at[pl.ds(0, ne)])

                    lane = lax.iota(jnp.int32, 16)

                    @pl.loop(0, ne // 16)
                    def _(g):
                        s16 = sblk[pl.ds(g * 16, 16)]
                        d16 = dblk[pl.ds(g * 16, 16)]
                        for k in range(16):
                            mk = lane == k
                            plsc.addupdate_scatter(cnt_a, [s16], ones, mask=mk)
                            plsc.addupdate_scatter(cnt_b, [d16], ones, mask=mk)

                tbase = sid * edges_per_tile

                @pl.loop(0, nfull)
                def _(blk):
                    hist_block(tbase + blk * DBLK, DBLK)

                if tail:
                    hist_block(tbase + nfull * DBLK, tail)

                for role in range(2):
                    cnt = cnt_a if role == 0 else cnt_b
                    n = n_s if role == 0 else n_d
                    o_ref = o_refs[2 * ri + role]
                    npad = _cpad(n)
                    stride = npad // 16
                    last = n - (NS - 1) * stride
                    pltpu.sync_copy(cnt.at[pl.ds(0, npad)],
                                    shared.at[pl.ds(sid * CMAX, npad)])
                    plsc.subcore_barrier()

                    @pl.loop(0, stride // 16)
                    def _(i):
                        ssum[pl.ds(i * 16, 16)] = zeros

                    for k in range(NS):
                        pltpu.sync_copy(
                            shared.at[pl.ds(k * CMAX + sid * stride, stride)],
                            srd.at[pl.ds(0, stride)])

                        @pl.loop(0, stride // 16)
                        def _(i):
                            sl = pl.ds(i * 16, 16)
                            ssum[sl] = ssum[sl] + srd[sl]

                    @pl.when(sid < NS - 1)
                    def _():
                        pltpu.sync_copy(ssum.at[pl.ds(0, stride)],
                                        o_ref.at[pl.ds(sid * stride, stride)])

                    @pl.when(sid == NS - 1)
                    def _():
                        pltpu.sync_copy(ssum.at[pl.ds(0, last)],
                                        o_ref.at[pl.ds(sid * stride, last)])
                    plsc.subcore_barrier()

    flat_in = []
    for r in rels:
        s1, d1, _, _ = edges[r]
        flat_in += [s1, d1]
    outs = body(*flat_in)
    return {r: (outs[2 * i], outs[2 * i + 1]) for i, r in enumerate(rels)}


def _matmul_call(x, degs_ws):
    n = x.shape[0]
    rows = n * T
    bn = 640
    assert rows % bn == 0 and (n + HPAD) * T % bn == 0
    grid = rows // bn
    x_flat = x.reshape(rows, D)
    nouts = len(degs_ws)

    def f(*refs):
        x_ref = refs[0]
        xv = x_ref[...]
        for i in range(nouts):
            deg_ref, w_ref, o_ref = refs[1 + 2 * i], refs[2 + 2 * i], refs[1 + 2 * nouts + i]
            rs = lax.rsqrt(jnp.maximum(deg_ref[...], 1.0))
            h = lax.dot_general(xv, w_ref[...], (((1,), (0,)), ((), ())),
                                preferred_element_type=jnp.float32,
                                precision=lax.Precision.HIGHEST)
            o_ref[...] = h * rs

    in_specs = [pl.BlockSpec((bn, D), lambda i: (i, 0))]
    inputs = [x_flat]
    for deg, w in degs_ws:
        deg_exp = jnp.repeat(deg, T)[:, None]
        in_specs.append(pl.BlockSpec((bn, 1), lambda i: (i, 0)))
        in_specs.append(pl.BlockSpec((D, H), lambda i: (0, 0)))
        inputs += [deg_exp, w]
    out_specs = [pl.BlockSpec((bn, H), lambda i: (i, 0))] * nouts
    out_shape = [jax.ShapeDtypeStruct(((n + HPAD) * T, H), jnp.float32)] * nouts

    outs = pl.pallas_call(f, grid=(grid,), in_specs=in_specs,
                          out_specs=out_specs, out_shape=out_shape)(*inputs)
    outs = outs if isinstance(outs, (list, tuple)) else [outs]
    return [o.reshape(n + HPAD, TH) for o in outs]


def _scatter_kernel(parts):
    rels = ["in", "ni", "ii", "sc"]

    out_type = [jax.ShapeDtypeStruct((_accr(parts[r][3]), TH), jnp.float32)
                for r in rels]

    @functools.partial(
        pl.kernel, out_type=out_type, mesh=_mesh, compiler_params=_sc_params,
        scratch_types=[
            pltpu.VMEM((EBLK,), jnp.int32),
            pltpu.VMEM((EBLK,), jnp.int32),
            pltpu.VMEM((FB,), jnp.int32),
            pltpu.VMEM((FB,), jnp.int32),
            pltpu.VMEM((GB, TH), jnp.float32),
            pltpu.VMEM((CROWS + 1, TH), jnp.float32),
            pltpu.SMEM((1,), jnp.int32),
            pltpu.SMEM((GB,), jnp.int32),
        ],
    )
    def body(*refs):
        in_refs = refs[: 3 * len(rels)]
        o_refs = refs[3 * len(rels): 4 * len(rels)]
        sblk, dblk, fs, fd, stage, chunk, wp, fdm = refs[4 * len(rels):]
        cid = lax.axis_index("core")
        sid = lax.axis_index("subcore")
        w = cid * NS + sid
        zeros = jnp.zeros((L,), jnp.float32)
        lane = lax.iota(jnp.int32, 16)
        trash16 = jnp.full((L,), CROWS, jnp.int32)
        zeros_i = jnp.zeros((L,), jnp.int32)

        for ri, r in enumerate(rels):
            h_ref = in_refs[3 * ri]
            src_r = in_refs[3 * ri + 1]
            dst_r = in_refs[3 * ri + 2]
            acc_ref = o_refs[ri]
            accr = acc_ref.shape[0]
            npass = accr // CAP
            ep = src_r.shape[0]
            nfull = ep // EBLK
            assert ep % EBLK == 0

            def fire0(h_ref=h_ref):
                sv = fs[pl.ds(0, GB)]
                pltpu.sync_copy(h_ref.at[sv], stage)
                dv = fd[pl.ds(0, GB)]
                for e in range(GB):
                    dloc = dv[e]

                    @pl.loop(0, 16)
                    def _(q):
                        for c2 in range(2):
                            sl = pl.ds(q * 32 + c2 * 16, 16)
                            chunk[dloc, sl] = chunk[dloc, sl] + stage[e, sl]

            @pl.loop(0, npass)
            def _(ps):
                lo = ps * CAP + w * CROWS

                @pl.loop(0, CROWS + 1)
                def _(rr):
                    for cg in range(TH // 16):
                        chunk[rr, pl.ds(cg * 16, 16)] = zeros

                wp[0] = 0

                def scan_block(base, ne):
                    pltpu.sync_copy(src_r.at[pl.ds(base, ne)],
                                    sblk.at[pl.ds(0, ne)])
                    pltpu.sync_copy(dst_r.at[pl.ds(base, ne)],
                                    dblk.at[pl.ds(0, ne)])

                    @pl.loop(0, ne // 16)
                    def _(g):
                        d16 = dblk[pl.ds(g * 16, 16)]
                        m = (d16 >= lo) & (d16 < lo + CROWS)

                        @pl.when(jnp.any(m))
                        def _():
                            s16 = sblk[pl.ds(g * 16, 16)]
                            mi = m.astype(jnp.int32)
                            p0 = wp[0]
                            pos = p0 + jnp.cumsum(mi) - 1
                            plsc.store_scatter(fs, [pos], s16, mask=m)
                            plsc.store_scatter(fd, [pos], d16 - lo, mask=m)
                            p1 = p0 + jnp.sum(mi)

                            @pl.when(p1 >= GB)
                            def _():
                                fire0()
                                fs[pl.ds(0, GB)] = fs[pl.ds(GB, GB)]
                                fd[pl.ds(0, GB)] = fd[pl.ds(GB, GB)]
                            wp[0] = jnp.where(p1 >= GB, p1 - GB, p1)

                @pl.loop(0, nfull)
                def _(blk):
                    scan_block(blk * EBLK, EBLK)

                rem = wp[0]
                pad_pos = rem + lane
                plsc.store_scatter(fs, [pad_pos], zeros_i)
                plsc.store_scatter(fd, [pad_pos], trash16)

                @pl.when(rem > 0)
                def _():
                    fire0()

                pltpu.sync_copy(chunk.at[pl.ds(0, CROWS)],
                                acc_ref.at[pl.ds(lo, CROWS)])

    flat_in = []
    for r in rels:
        h, s1, d1, _ = parts[r]
        flat_in += [h, s1, d1]
    outs = body(*flat_in)
    return dict(zip(rels, outs))


def _epilogue_call(parts):
    n = parts[0][1].shape[0]
    bn = 1200 if n % 1200 == 0 else 1000
    assert n % bn == 0
    grid = n // bn
    nr = len(parts)
    scale = 1.0 / nr

    def f(*refs):
        o_ref = refs[3 * nr]
        acc = jnp.zeros((bn, TH), jnp.float32)
        for i in range(nr):
            a_ref, deg_ref, b_ref = refs[3 * i], refs[3 * i + 1], refs[3 * i + 2]
            rs = lax.rsqrt(jnp.maximum(deg_ref[...], 1.0))
            acc = acc + (a_ref[...] * rs + b_ref[...][None, :]) * scale
        o_ref[...] = jnp.where(acc >= 0, acc, 0.01 * acc)

    in_specs, inputs = [], []
    for a, deg, b in parts:
        in_specs.append(pl.BlockSpec((bn, TH), lambda i: (i, 0)))
        in_specs.append(pl.BlockSpec((bn, 1), lambda i: (i, 0)))
        in_specs.append(pl.BlockSpec((TH,), lambda i: (0,)))
        inputs += [a, deg[:, None], jnp.tile(b, T)]

    out = pl.pallas_call(
        f, grid=(grid,), in_specs=in_specs,
        out_specs=pl.BlockSpec((bn, TH), lambda i: (i, 0)),
        out_shape=jax.ShapeDtypeStruct((n, TH), jnp.float32))(*inputs)
    return out.reshape(n, T, H)


def kernel(node_feat, pod_feat, svc_feat,
           W_sc, b_sc, W_in, b_in, W_ni, b_ni, W_ii, b_ii,
           e_sc, e_in, e_ni, e_ii):
    edges = {
        "in": _pad_edges(e_in, POD_N, NODE_N) + (POD_N, NODE_N),
        "ni": _pad_edges(e_ni, NODE_N, POD_N) + (NODE_N, POD_N),
        "ii": _pad_edges(e_ii, POD_N, POD_N) + (POD_N, POD_N),
        "sc": _pad_edges(e_sc, SVC_N, SVC_N) + (SVC_N, SVC_N),
    }
    degs = _degree_kernel(edges)

    h_in, h_ii = _matmul_call(pod_feat, [(degs["in"][0], W_in), (degs["ii"][0], W_ii)])
    (h_ni,) = _matmul_call(node_feat, [(degs["ni"][0], W_ni)])
    (h_sc,) = _matmul_call(svc_feat, [(degs["sc"][0], W_sc)])

    accs = _scatter_kernel(
        {"in": (h_in, edges["in"][0], edges["in"][1], NODE_N),
         "ni": (h_ni, edges["ni"][0], edges["ni"][1], POD_N),
         "ii": (h_ii, edges["ii"][0], edges["ii"][1], POD_N),
         "sc": (h_sc, edges["sc"][0], edges["sc"][1], SVC_N)})

    node_out = _epilogue_call([(accs["in"], degs["in"][1], b_in)])
    pod_out = _epilogue_call([(accs["ni"], degs["ni"][1], b_ni),
                              (accs["ii"], degs["ii"][1], b_ii)])
    svc_out = _epilogue_call([(accs["sc"], degs["sc"][1], b_sc)])

    return jnp.concatenate([node_out, pod_out, svc_out], axis=0)


# speedup vs baseline: 1.0274x; 1.0274x over previous
import dataclasses
import functools

import jax
import jax.numpy as jnp
from jax import lax
from jax.experimental import pallas as pl
from jax.experimental.pallas import tpu as pltpu
from jax.experimental.pallas import tpu_sc as plsc

NODE_N = 10000
POD_N = 30000
SVC_N = 10000
T = 8
D = 64
H = 64
TH = T * H

NC = 2
NS = 16
L = 16

BATCH = 128
PADR = 64
HPAD = 80
CROWS = 192
CAP = NC * NS * CROWS
EBLK = 2048
GB = 16
FB = 48

_mesh = plsc.VectorSubcoreMesh(core_axis_name="core", subcore_axis_name="subcore")

_sc_params = pltpu.CompilerParams()
if "needs_layout_passes" in pltpu.CompilerParams.__dataclass_fields__:
    _sc_params = dataclasses.replace(_sc_params, needs_layout_passes=False)


def _cpad(n):
    return ((n + PADR + 255) // 256) * 256


def _accr(n):
    return ((n + PADR + CAP - 1) // CAP) * CAP


def _pad_edges(e, n_src, n_dst):
    E = e.shape[1]
    ep = ((E + NS * BATCH - 1) // (NS * BATCH)) * (NS * BATCH)
    fan = (jnp.arange(ep - E, dtype=jnp.int32) % PADR)
    src = jnp.concatenate([e[0].astype(jnp.int32), n_src + fan])
    dst = jnp.concatenate([e[1].astype(jnp.int32), n_dst + fan])
    return src, dst


def _degree_kernel(edges):
    rels = ["in", "ni", "ii", "sc"]
    core_of = {"in": 0, "ni": 0, "ii": 1, "sc": 1}
    CMAX = _cpad(POD_N)
    DBLK = 128

    out_type = []
    for r in rels:
        _, _, n_s, n_d = edges[r]
        out_type.append(jax.ShapeDtypeStruct((n_s,), jnp.float32))
        out_type.append(jax.ShapeDtypeStruct((n_d,), jnp.float32))

    @functools.partial(
        pl.kernel, out_type=out_type, mesh=_mesh, compiler_params=_sc_params,
        scratch_types=[
            pltpu.VMEM((DBLK,), jnp.int32),
            pltpu.VMEM((DBLK,), jnp.int32),
            pltpu.VMEM((CMAX,), jnp.float32),
            pltpu.VMEM((CMAX,), jnp.float32),
            pltpu.VMEM((CMAX // 16,), jnp.float32),
            pltpu.VMEM((CMAX // 16,), jnp.float32),
            pltpu.VMEM_SHARED((NS * CMAX,), jnp.float32),
        ],
    )
    def body(*refs):
        in_refs = refs[: 2 * len(rels)]
        o_refs = refs[2 * len(rels): 4 * len(rels)]
        sblk, dblk, cnt_a, cnt_b, ssum, srd, shared = refs[4 * len(rels):]
        cid = lax.axis_index("core")
        sid = lax.axis_index("subcore")
        ones = jnp.full((L,), 1.0, jnp.float32)
        zeros = jnp.zeros((L,), jnp.float32)

        for ri, r in enumerate(rels):
            _, _, n_s, n_d = edges[r]
            src_r = in_refs[2 * ri]
            dst_r = in_refs[2 * ri + 1]
            edges_per_tile = src_r.shape[0] // NS
            nfull = edges_per_tile // DBLK
            tail = edges_per_tile % DBLK

            @pl.when(cid == core_of[r])
            def _():
                @pl.loop(0, _cpad(n_s) // 16)
                def _(i):
                    cnt_a[pl.ds(i * 16, 16)] = zeros

                @pl.loop(0, _cpad(n_d) // 16)
                def _(i):
                    cnt_b[pl.ds(i * 16, 16)] = zeros

                def hist_block(base, ne):
                    pltpu.sync_copy(src_r.at[pl.ds(base, ne)], sblk.at[pl.ds(0, ne)])
                    pltpu.sync_copy(dst_r.at[pl.ds(base, ne)], dblk.at[pl.ds(0, ne)])

                    lane = lax.iota(jnp.int32, 16)

                    @pl.loop(0, ne // 16)
                    def _(g):
                        s16 = sblk[pl.ds(g * 16, 16)]
                        d16 = dblk[pl.ds(g * 16, 16)]
                        for k in range(16):
                            mk = lane == k
                            plsc.addupdate_scatter(cnt_a, [s16], ones, mask=mk)
                            plsc.addupdate_scatter(cnt_b, [d16], ones, mask=mk)

                tbase = sid * edges_per_tile

                @pl.loop(0, nfull)
                def _(blk):
                    hist_block(tbase + blk * DBLK, DBLK)

                if tail:
                    hist_block(tbase + nfull * DBLK, tail)

                for role in range(2):
                    cnt = cnt_a if role == 0 else cnt_b
                    n = n_s if role == 0 else n_d
                    o_ref = o_refs[2 * ri + role]
                    npad = _cpad(n)
                    stride = npad // 16
                    last = n - (NS - 1) * stride
                    pltpu.sync_copy(cnt.at[pl.ds(0, npad)],
                                    shared.at[pl.ds(sid * CMAX, npad)])
                    plsc.subcore_barrier()

                    @pl.loop(0, stride // 16)
                    def _(i):
                        ssum[pl.ds(i * 16, 16)] = zeros

                    for k in range(NS):
                        pltpu.sync_copy(
                            shared.at[pl.ds(k * CMAX + sid * stride, stride)],
                            srd.at[pl.ds(0, stride)])

                        @pl.loop(0, stride // 16)
                        def _(i):
                            sl = pl.ds(i * 16, 16)
                            ssum[sl] = ssum[sl] + srd[sl]

                    @pl.when(sid < NS - 1)
                    def _():
                        pltpu.sync_copy(ssum.at[pl.ds(0, stride)],
                                        o_ref.at[pl.ds(sid * stride, stride)])

                    @pl.when(sid == NS - 1)
                    def _():
                        pltpu.sync_copy(ssum.at[pl.ds(0, last)],
                                        o_ref.at[pl.ds(sid * stride, last)])
                    plsc.subcore_barrier()

    flat_in = []
    for r in rels:
        s1, d1, _, _ = edges[r]
        flat_in += [s1, d1]
    outs = body(*flat_in)
    return {r: (outs[2 * i], outs[2 * i + 1]) for i, r in enumerate(rels)}


def _matmul_call(x, degs_ws):
    n = x.shape[0]
    rows = n * T
    bn = 640
    assert rows % bn == 0 and (n + HPAD) * T % bn == 0
    grid = rows // bn
    x_flat = x.reshape(rows, D)
    nouts = len(degs_ws)

    def f(*refs):
        x_ref = refs[0]
        xv = x_ref[...]
        for i in range(nouts):
            deg_ref, w_ref, o_ref = refs[1 + 2 * i], refs[2 + 2 * i], refs[1 + 2 * nouts + i]
            rs = lax.rsqrt(jnp.maximum(deg_ref[...], 1.0))
            h = lax.dot_general(xv, w_ref[...], (((1,), (0,)), ((), ())),
                                preferred_element_type=jnp.float32,
                                precision=lax.Precision.HIGHEST)
            o_ref[...] = h * rs

    in_specs = [pl.BlockSpec((bn, D), lambda i: (i, 0))]
    inputs = [x_flat]
    for deg, w in degs_ws:
        deg_exp = jnp.repeat(deg, T)[:, None]
        in_specs.append(pl.BlockSpec((bn, 1), lambda i: (i, 0)))
        in_specs.append(pl.BlockSpec((D, H), lambda i: (0, 0)))
        inputs += [deg_exp, w]
    out_specs = [pl.BlockSpec((bn, H), lambda i: (i, 0))] * nouts
    out_shape = [jax.ShapeDtypeStruct(((n + HPAD) * T, H), jnp.float32)] * nouts

    outs = pl.pallas_call(f, grid=(grid,), in_specs=in_specs,
                          out_specs=out_specs, out_shape=out_shape)(*inputs)
    outs = outs if isinstance(outs, (list, tuple)) else [outs]
    return [o.reshape(n + HPAD, TH) for o in outs]


def _scatter_kernel(parts):
    rels = ["in", "ni", "ii", "sc"]

    out_type = [jax.ShapeDtypeStruct((_accr(parts[r][3]), TH), jnp.float32)
                for r in rels]

    @functools.partial(
        pl.kernel, out_type=out_type, mesh=_mesh, compiler_params=_sc_params,
        scratch_types=[
            pltpu.VMEM((EBLK,), jnp.int32),
            pltpu.VMEM((EBLK,), jnp.int32),
            pltpu.VMEM((FB,), jnp.int32),
            pltpu.VMEM((FB,), jnp.int32),
            pltpu.VMEM((GB, TH), jnp.float32),
            pltpu.VMEM((CROWS + 1, TH), jnp.float32),
            pltpu.SMEM((1,), jnp.int32),
        ],
    )
    def body(*refs):
        in_refs = refs[: 3 * len(rels)]
        o_refs = refs[3 * len(rels): 4 * len(rels)]
        sblk, dblk, fs, fd, stage, chunk, wp = refs[4 * len(rels):]
        cid = lax.axis_index("core")
        sid = lax.axis_index("subcore")
        w = cid * NS + sid
        zeros = jnp.zeros((L,), jnp.float32)
        lane = lax.iota(jnp.int32, 16)
        trash16 = jnp.full((L,), CROWS, jnp.int32)
        zeros_i = jnp.zeros((L,), jnp.int32)

        for ri, r in enumerate(rels):
            h_ref = in_refs[3 * ri]
            src_r = in_refs[3 * ri + 1]
            dst_r = in_refs[3 * ri + 2]
            acc_ref = o_refs[ri]
            accr = acc_ref.shape[0]
            npass = accr // CAP
            ep = src_r.shape[0]
            nfull = ep // EBLK
            assert ep % EBLK == 0

            def fire0(h_ref=h_ref):
                sv = fs[pl.ds(0, GB)]
                pltpu.sync_copy(h_ref.at[sv], stage)
                dv = fd[pl.ds(0, GB)]
                for e in range(GB):
                    dloc = dv[e]

                    @pl.loop(0, 16)
                    def _(q):
                        for c2 in range(2):
                            sl = pl.ds(q * 32 + c2 * 16, 16)
                            chunk[dloc, sl] = chunk[dloc, sl] + stage[e, sl]

            @pl.loop(0, npass)
            def _(ps):
                lo = ps * CAP + w * CROWS

                @pl.loop(0, CROWS + 1)
                def _(rr):
                    for cg in range(TH // 16):
                        chunk[rr, pl.ds(cg * 16, 16)] = zeros

                wp[0] = 0

                def scan_block(base, ne):
                    pltpu.sync_copy(src_r.at[pl.ds(base, ne)],
                                    sblk.at[pl.ds(0, ne)])
                    pltpu.sync_copy(dst_r.at[pl.ds(base, ne)],
                                    dblk.at[pl.ds(0, ne)])

                    @pl.loop(0, ne // 16)
                    def _(g):
                        d16 = dblk[pl.ds(g * 16, 16)]
                        m = (d16 >= lo) & (d16 < lo + CROWS)

                        @pl.when(jnp.any(m))
                        def _():
                            s16 = sblk[pl.ds(g * 16, 16)]
                            mi = m.astype(jnp.int32)
                            p0 = wp[0]
                            pos = p0 + jnp.cumsum(mi) - 1
                            plsc.store_scatter(fs, [pos], s16, mask=m)
                            plsc.store_scatter(fd, [pos], d16 - lo, mask=m)
                            p1 = p0 + jnp.sum(mi)

                            @pl.when(p1 >= GB)
                            def _():
                                fire0()
                                fs[pl.ds(0, GB)] = fs[pl.ds(GB, GB)]
                                fd[pl.ds(0, GB)] = fd[pl.ds(GB, GB)]
                            wp[0] = jnp.where(p1 >= GB, p1 - GB, p1)

                @pl.loop(0, nfull)
                def _(blk):
                    scan_block(blk * EBLK, EBLK)

                rem = wp[0]
                pad_pos = rem + lane
                plsc.store_scatter(fs, [pad_pos], zeros_i)
                plsc.store_scatter(fd, [pad_pos], trash16)

                @pl.when(rem > 0)
                def _():
                    fire0()

                pltpu.sync_copy(chunk.at[pl.ds(0, CROWS)],
                                acc_ref.at[pl.ds(lo, CROWS)])

    flat_in = []
    for r in rels:
        h, s1, d1, _ = parts[r]
        flat_in += [h, s1, d1]
    outs = body(*flat_in)
    return dict(zip(rels, outs))


def _epilogue_call(parts):
    n = parts[0][1].shape[0]
    bn = 1200 if n % 1200 == 0 else 1000
    assert n % bn == 0
    grid = n // bn
    nr = len(parts)
    scale = 1.0 / nr

    def f(*refs):
        o_ref = refs[3 * nr]
        acc = jnp.zeros((bn, TH), jnp.float32)
        for i in range(nr):
            a_ref, deg_ref, b_ref = refs[3 * i], refs[3 * i + 1], refs[3 * i + 2]
            rs = lax.rsqrt(jnp.maximum(deg_ref[...], 1.0))
            acc = acc + (a_ref[...] * rs + b_ref[...][None, :]) * scale
        o_ref[...] = jnp.where(acc >= 0, acc, 0.01 * acc)

    in_specs, inputs = [], []
    for a, deg, b in parts:
        in_specs.append(pl.BlockSpec((bn, TH), lambda i: (i, 0)))
        in_specs.append(pl.BlockSpec((bn, 1), lambda i: (i, 0)))
        in_specs.append(pl.BlockSpec((TH,), lambda i: (0,)))
        inputs += [a, deg[:, None], jnp.tile(b, T)]

    out = pl.pallas_call(
        f, grid=(grid,), in_specs=in_specs,
        out_specs=pl.BlockSpec((bn, TH), lambda i: (i, 0)),
        out_shape=jax.ShapeDtypeStruct((n, TH), jnp.float32))(*inputs)
    return out.reshape(n, T, H)


def kernel(node_feat, pod_feat, svc_feat,
           W_sc, b_sc, W_in, b_in, W_ni, b_ni, W_ii, b_ii,
           e_sc, e_in, e_ni, e_ii):
    edges = {
        "in": _pad_edges(e_in, POD_N, NODE_N) + (POD_N, NODE_N),
        "ni": _pad_edges(e_ni, NODE_N, POD_N) + (NODE_N, POD_N),
        "ii": _pad_edges(e_ii, POD_N, POD_N) + (POD_N, POD_N),
        "sc": _pad_edges(e_sc, SVC_N, SVC_N) + (SVC_N, SVC_N),
    }
    degs = _degree_kernel(edges)

    h_in, h_ii = _matmul_call(pod_feat, [(degs["in"][0], W_in), (degs["ii"][0], W_ii)])
    (h_ni,) = _matmul_call(node_feat, [(degs["ni"][0], W_ni)])
    (h_sc,) = _matmul_call(svc_feat, [(degs["sc"][0], W_sc)])

    accs = _scatter_kernel(
        {"in": (h_in, edges["in"][0], edges["in"][1], NODE_N),
         "ni": (h_ni, edges["ni"][0], edges["ni"][1], POD_N),
         "ii": (h_ii, edges["ii"][0], edges["ii"][1], POD_N),
         "sc": (h_sc, edges["sc"][0], edges["sc"][1], SVC_N)})

    node_out = _epilogue_call([(accs["in"], degs["in"][1], b_in)])
    pod_out = _epilogue_call([(accs["ni"], degs["ni"][1], b_ni),
                              (accs["ii"], degs["ii"][1], b_ii)])
    svc_out = _epilogue_call([(accs["sc"], degs["sc"][1], b_sc)])

    return jnp.concatenate([node_out, pod_out, svc_out], axis=0)
